# baseline (speedup 1.0000x reference)
import functools

import jax
import jax.numpy as jnp
from jax import lax
from jax.experimental import pallas as pl
from jax.experimental.pallas import tpu as pltpu
from jax.experimental.pallas import tpu_sc as plsc

D = 128
T = 4
N_GRAPHS = 64
RU = 256

NC, NS = 2, 16
NW = NC * NS
CHUNK = 80

_SELU_ALPHA = 1.6732632423543772
_SELU_SCALE = 1.0507009873554805


def _selu(x):
    neg = _SELU_SCALE * _SELU_ALPHA * (jnp.exp(jnp.minimum(x, 0.0)) - 1.0)
    return jnp.where(x > 0.0, _SELU_SCALE * x, neg)


def _ab_body(h_ref, w1_ref, w2_ref, bm_ref, ab_ref):
    h = h_ref[...]
    ab_ref[0] = jnp.dot(h, w1_ref[...], preferred_element_type=jnp.float32)
    ab_ref[1] = (
        jnp.dot(h, w2_ref[...], preferred_element_type=jnp.float32) + bm_ref[...]
    )


def _ab_call(h, w1, w2, bm):
    n = h.shape[0]
    blk = 1000
    grid = (n // blk,)
    return pl.pallas_call(
        _ab_body,
        grid=grid,
        in_specs=[
            pl.BlockSpec((blk, D), lambda i: (i, 0)),
            pl.BlockSpec((D, D), lambda i: (0, 0)),
            pl.BlockSpec((D, D), lambda i: (0, 0)),
            pl.BlockSpec((1, D), lambda i: (0, 0)),
        ],
        out_specs=pl.BlockSpec((2, blk, D), lambda i: (0, i, 0)),
        out_shape=jax.ShapeDtypeStruct((2, n, D), jnp.float32),
    )(h, w1, w2, bm)


def _gru_body(s_ref, h_ref, wg_ref, ug_ref, bi_ref, br_ref, o_ref):
    x = s_ref[0] + s_ref[1]
    h = h_ref[...]
    mx = jnp.dot(x, wg_ref[...], preferred_element_type=jnp.float32) + bi_ref[...]
    mh = jnp.dot(h, ug_ref[...], preferred_element_type=jnp.float32) + br_ref[...]
    z = jax.nn.sigmoid(mx[:, :D] + mh[:, :D])
    r = jax.nn.sigmoid(mx[:, D : 2 * D] + mh[:, D : 2 * D])
    cand = jnp.tanh(mx[:, 2 * D :] + r * mh[:, 2 * D :])
    o_ref[...] = z * h + (1.0 - z) * cand


def _gru_call(s, h, wg, ug, bi, br):
    n = h.shape[0]
    blk = 1000
    grid = (n // blk,)
    return pl.pallas_call(
        _gru_body,
        grid=grid,
        in_specs=[
            pl.BlockSpec((NC, blk, D), lambda i: (0, i, 0)),
            pl.BlockSpec((blk, D), lambda i: (i, 0)),
            pl.BlockSpec((D, 3 * D), lambda i: (0, 0)),
            pl.BlockSpec((D, 3 * D), lambda i: (0, 0)),
            pl.BlockSpec((1, 3 * D), lambda i: (0, 0)),
            pl.BlockSpec((1, 3 * D), lambda i: (0, 0)),
        ],
        out_specs=pl.BlockSpec((blk, D), lambda i: (i, 0)),
        out_shape=jax.ShapeDtypeStruct((n, D), jnp.float32),
    )(s, h, wg, ug, bi, br)


def _readout_body(h_ref, gid_ref, w1_ref, b1_ref, w2_ref, b2_ref, w3_ref, b3_ref, o_ref):
    n = h_ref.shape[0]
    ids = gid_ref[...]
    iota = lax.broadcasted_iota(jnp.int32, (N_GRAPHS, n), 0)
    mask = (ids == iota).astype(jnp.float32)
    gemb = jnp.dot(mask, h_ref[...], preferred_element_type=jnp.float32)
    r1 = _selu(jnp.dot(gemb, w1_ref[...], preferred_element_type=jnp.float32) + b1_ref[...])
    r2 = _selu(jnp.dot(r1, w2_ref[...], preferred_element_type=jnp.float32) + b2_ref[...])
    o_ref[...] = jnp.sum(r2 * w3_ref[...], axis=1, keepdims=True) + b3_ref[...]


def _readout_call(h, gid, w1, b1, w2, b2, w3t, b3):
    n = h.shape[0]
    return pl.pallas_call(
        _readout_body,
        in_specs=[
            pl.BlockSpec((n, D), lambda: (0, 0)),
            pl.BlockSpec((1, n), lambda: (0, 0)),
            pl.BlockSpec((D, RU), lambda: (0, 0)),
            pl.BlockSpec((1, RU), lambda: (0, 0)),
            pl.BlockSpec((RU, RU), lambda: (0, 0)),
            pl.BlockSpec((1, RU), lambda: (0, 0)),
            pl.BlockSpec((1, RU), lambda: (0, 0)),
            pl.BlockSpec((1, 1), lambda: (0, 0)),
        ],
        out_specs=pl.BlockSpec((N_GRAPHS, 1), lambda: (0, 0)),
        out_shape=jax.ShapeDtypeStruct((N_GRAPHS, 1), jnp.float32),
    )(h, gid, w1, b1, w2, b2, w3t, b3)


def _edge_body(ab_hbm, idxc_hbm, out_hbm, win, mbuf, fbuf, fidx, s_sh, sem):
    n = ab_hbm.shape[1]
    sb = 80
    nblk = n // sb
    nchunk = idxc_hbm.shape[1] // 2
    nown = (nblk + NS - 1) // NS
    WROWS = win.shape[0]

    cid = lax.axis_index("c")
    sid = lax.axis_index("s")
    wid = cid * NS + sid

    def zrow(i, _):
        for c in range(D // 16):
            fbuf[i, pl.ds(c * 16, 16)] = jnp.zeros((16,), jnp.float32)
        return 0
    lax.fori_loop(0, fbuf.shape[0], zrow, 0)

    def zcp(i, _):
        j = i * NS + sid

        @pl.when(j < nblk)
        def _():
            pltpu.sync_copy(fbuf.at[pl.ds(0, sb)], s_sh.at[pl.ds(j * sb, sb)])
        return 0
    lax.fori_loop(0, nown, zcp, 0)
    plsc.subcore_barrier()

    trash16 = jnp.full((16,), n, jnp.int32)
    for k in range(fidx.shape[1] // 16):
        fidx[0, pl.ds(k * 16, 16)] = trash16

    lane_iota = lax.iota(jnp.int32, 16)

    def _set_fidx(pos, val):
        base = (pos // 16) * 16
        lane = pos - base
        v = fidx[0, pl.ds(base, 16)]
        fidx[0, pl.ds(base, 16)] = jnp.where(lane_iota == lane, val, v)

    def chunk(j, carry):
        prevdst, fcnt = carry
        jj = j % (WROWS // 2)

        @pl.when(jj == 0)
        def _():
            pltpu.sync_copy(idxc_hbm.at[wid, pl.ds(2 * j, WROWS)], win)

        def pair(q, _):
            pltpu.async_copy(
                ab_hbm.at[q].at[win.at[2 * jj + q]],
                mbuf.at[pl.ds(1 + q * 87, CHUNK)],
                sem,
            ).wait()
            return 0
        lax.fori_loop(0, 2, pair, 0)

        def ew(e, _):
            for c in range(D // 16):
                sl = pl.ds(c * 16, 16)
                x = mbuf[e, sl] + mbuf[87 + e, sl]
                neg = _SELU_ALPHA * (jnp.exp(jnp.minimum(x, 0.0)) - 1.0)
                mbuf[e, sl] = jnp.where(x > 0.0, x, neg) * _SELU_SCALE
            return 0
        lax.fori_loop(1, CHUNK + 1, ew, 0)

        is_last = j == nchunk - 1

        def fold(g, fc):
            pdst, cnt = fc
            idxv = win[2 * jj + 1, pl.ds(g * 16, 16)]
            for k in range(16):
                e = g * 16 + k + 1
                dst = jnp.where(is_last, n, idxv[k])
                same = dst == pdst

                @pl.when(same)
                def _():
                    for c in range(D // 16):
                        sl = pl.ds(c * 16, 16)
                        mbuf[e, sl] = mbuf[e, sl] + mbuf[e - 1, sl]

                @pl.when(jnp.logical_not(same))
                def _():
                    for c in range(D // 16):
                        sl = pl.ds(c * 16, 16)
                        fbuf[cnt, sl] = mbuf[e - 1, sl]
                    _set_fidx(cnt, pdst)

                cnt = jnp.where(same, cnt, cnt + 1)
                pdst = dst

            do_flush = jnp.logical_or(
                cnt >= 80, jnp.logical_and(is_last, g == CHUNK // 16 - 1)
            )

            @pl.when(do_flush)
            def _():
                pltpu.sync_copy(fbuf, s_sh.at[fidx.at[0]], add=True)
                for k in range(fidx.shape[1] // 16):
                    fidx[0, pl.ds(k * 16, 16)] = trash16

            cnt = jnp.where(do_flush, 0, cnt)
            return pdst, cnt
        prevdst, fcnt = lax.fori_loop(0, CHUNK // 16, fold, (prevdst, fcnt))

        for c in range(D // 16):
            sl = pl.ds(c * 16, 16)
            mbuf[0, sl] = mbuf[CHUNK, sl]
        return prevdst, fcnt

    lax.fori_loop(0, nchunk, chunk, (n, 0))
    plsc.subcore_barrier()

    def ocp(i, _):
        j = i * NS + sid

        @pl.when(j < nblk)
        def _():
            rows = pl.ds(j * sb, sb)
            pltpu.sync_copy(s_sh.at[rows], fbuf.at[pl.ds(0, sb)])
            pltpu.sync_copy(fbuf.at[pl.ds(0, sb)], out_hbm.at[cid, rows])
        return 0
    lax.fori_loop(0, nown, ocp, 0)


def _edge_call(ab, idxc):
    n = ab.shape[1]
    nchunk2 = idxc.shape[1]
    mesh = plsc.VectorSubcoreMesh(
        core_axis_name="c", subcore_axis_name="s", num_cores=NC, num_subcores=NS
    )
    fn = pl.kernel(
        _edge_body,
        out_type=jax.ShapeDtypeStruct((NC, n, D), jnp.float32),
        mesh=mesh,
        compiler_params=pltpu.CompilerParams(use_tc_tiling_on_sc=False),
        scratch_types=[
            pltpu.VMEM((36, CHUNK), jnp.int32),
            pltpu.VMEM((168, D), jnp.float32),
            pltpu.VMEM((96, D), jnp.float32),
            pltpu.VMEM((1, 96), jnp.int32),
            pltpu.VMEM_SHARED((n + 8, D), jnp.float32),
            pltpu.SemaphoreType.DMA,
        ],
    )
    return fn(ab, idxc)


def kernel(link_state, W_msg, b_msg, W_gru, U_gru, b_gru_in, b_gru_rec,
           W_r1, b_r1, W_r2, b_r2, W_r3, b_r3,
           states_graph_ids, states_first, states_second, sates_num_edges):
    n = link_state.shape[0]
    e = states_first.shape[0]
    assert e % (NW * CHUNK) == 0 and n % NS == 0

    w1 = W_msg[:D]
    w2 = W_msg[D:]
    bm = b_msg.reshape(1, D)
    bi = b_gru_in.reshape(1, 3 * D)
    br = b_gru_rec.reshape(1, 3 * D)
    perm = jnp.argsort(states_second, stable=True)
    sf = jnp.take(states_first, perm)
    ss = jnp.take(states_second, perm)

    nchunk = e // (NW * CHUNK)
    f3d = sf.reshape(NW, nchunk, 1, CHUNK)
    s3d = ss.reshape(NW, nchunk, 1, CHUNK)
    idxc = jnp.concatenate([f3d, s3d], axis=2).reshape(NW, 2 * nchunk, CHUNK)
    idxc = jnp.concatenate(
        [idxc, jnp.zeros((NW, 2, CHUNK), jnp.int32)], axis=1
    )
    gid = states_graph_ids.reshape(1, n)

    h = link_state
    for _ in range(T):
        ab = _ab_call(h, w1, w2, bm)
        s = _edge_call(ab, idxc)
        h = _gru_call(s, h, W_gru, U_gru, bi, br)

    return _readout_call(
        h, gid,
        W_r1, b_r1.reshape(1, RU),
        W_r2, b_r2.reshape(1, RU),
        W_r3.reshape(1, RU), b_r3.reshape(1, 1),
    )


# speedup vs baseline: 1.6740x; 1.6740x over previous
import functools

import jax
import jax.numpy as jnp
from jax import lax
from jax.experimental import pallas as pl
from jax.experimental.pallas import tpu as pltpu
from jax.experimental.pallas import tpu_sc as plsc

D = 128
T = 4
N_GRAPHS = 64
RU = 256

NC, NS = 2, 16
NW = NC * NS
CHUNK = 80

_SELU_ALPHA = 1.6732632423543772
_SELU_SCALE = 1.0507009873554805


def _selu(x):
    neg = _SELU_SCALE * _SELU_ALPHA * (jnp.exp(jnp.minimum(x, 0.0)) - 1.0)
    return jnp.where(x > 0.0, _SELU_SCALE * x, neg)


def _ab_body(h_ref, w1_ref, w2_ref, bm_ref, ab_ref):
    h = h_ref[...]
    ab_ref[0] = jnp.dot(h, w1_ref[...], preferred_element_type=jnp.float32)
    ab_ref[1] = (
        jnp.dot(h, w2_ref[...], preferred_element_type=jnp.float32) + bm_ref[...]
    )


def _ab_call(h, w1, w2, bm):
    n = h.shape[0]
    blk = 1000
    grid = (n // blk,)
    return pl.pallas_call(
        _ab_body,
        grid=grid,
        in_specs=[
            pl.BlockSpec((blk, D), lambda i: (i, 0)),
            pl.BlockSpec((D, D), lambda i: (0, 0)),
            pl.BlockSpec((D, D), lambda i: (0, 0)),
            pl.BlockSpec((1, D), lambda i: (0, 0)),
        ],
        out_specs=pl.BlockSpec((2, blk, D), lambda i: (0, i, 0)),
        out_shape=jax.ShapeDtypeStruct((2, n, D), jnp.float32),
    )(h, w1, w2, bm)


def _gru_body(s_ref, h_ref, wg_ref, ug_ref, bi_ref, br_ref, o_ref):
    x = s_ref[0] + s_ref[1]
    h = h_ref[...]
    mx = jnp.dot(x, wg_ref[...], preferred_element_type=jnp.float32) + bi_ref[...]
    mh = jnp.dot(h, ug_ref[...], preferred_element_type=jnp.float32) + br_ref[...]
    z = jax.nn.sigmoid(mx[:, :D] + mh[:, :D])
    r = jax.nn.sigmoid(mx[:, D : 2 * D] + mh[:, D : 2 * D])
    cand = jnp.tanh(mx[:, 2 * D :] + r * mh[:, 2 * D :])
    o_ref[...] = z * h + (1.0 - z) * cand


def _gru_call(s, h, wg, ug, bi, br):
    n = h.shape[0]
    blk = 1000
    grid = (n // blk,)
    return pl.pallas_call(
        _gru_body,
        grid=grid,
        in_specs=[
            pl.BlockSpec((NC, blk, D), lambda i: (0, i, 0)),
            pl.BlockSpec((blk, D), lambda i: (i, 0)),
            pl.BlockSpec((D, 3 * D), lambda i: (0, 0)),
            pl.BlockSpec((D, 3 * D), lambda i: (0, 0)),
            pl.BlockSpec((1, 3 * D), lambda i: (0, 0)),
            pl.BlockSpec((1, 3 * D), lambda i: (0, 0)),
        ],
        out_specs=pl.BlockSpec((blk, D), lambda i: (i, 0)),
        out_shape=jax.ShapeDtypeStruct((n, D), jnp.float32),
    )(s, h, wg, ug, bi, br)


def _readout_body(h_ref, gid_ref, w1_ref, b1_ref, w2_ref, b2_ref, w3_ref, b3_ref, o_ref):
    n = h_ref.shape[0]
    ids = gid_ref[...]
    iota = lax.broadcasted_iota(jnp.int32, (N_GRAPHS, n), 0)
    mask = (ids == iota).astype(jnp.float32)
    gemb = jnp.dot(mask, h_ref[...], preferred_element_type=jnp.float32)
    r1 = _selu(jnp.dot(gemb, w1_ref[...], preferred_element_type=jnp.float32) + b1_ref[...])
    r2 = _selu(jnp.dot(r1, w2_ref[...], preferred_element_type=jnp.float32) + b2_ref[...])
    o_ref[...] = jnp.sum(r2 * w3_ref[...], axis=1, keepdims=True) + b3_ref[...]


def _readout_call(h, gid, w1, b1, w2, b2, w3t, b3):
    n = h.shape[0]
    return pl.pallas_call(
        _readout_body,
        in_specs=[
            pl.BlockSpec((n, D), lambda: (0, 0)),
            pl.BlockSpec((1, n), lambda: (0, 0)),
            pl.BlockSpec((D, RU), lambda: (0, 0)),
            pl.BlockSpec((1, RU), lambda: (0, 0)),
            pl.BlockSpec((RU, RU), lambda: (0, 0)),
            pl.BlockSpec((1, RU), lambda: (0, 0)),
            pl.BlockSpec((1, RU), lambda: (0, 0)),
            pl.BlockSpec((1, 1), lambda: (0, 0)),
        ],
        out_specs=pl.BlockSpec((N_GRAPHS, 1), lambda: (0, 0)),
        out_shape=jax.ShapeDtypeStruct((N_GRAPHS, 1), jnp.float32),
    )(h, gid, w1, b1, w2, b2, w3t, b3)


def _edge_body(ab_hbm, idxc_hbm, out_hbm, idxc, mbuf, stage, s_sh, sem):
    n = ab_hbm.shape[1]
    sb = stage.shape[0]
    nblk = n // sb
    nchunk = idxc.shape[0] // 2
    nown = (nblk + NS - 1) // NS

    cid = lax.axis_index("c")
    sid = lax.axis_index("s")
    wid = cid * NS + sid

    def zrow(i, _):
        for c in range(D // 16):
            stage[i, pl.ds(c * 16, 16)] = jnp.zeros((16,), jnp.float32)
        return 0
    lax.fori_loop(0, sb, zrow, 0)

    def zcp(i, _):
        j = i * NS + sid

        @pl.when(j < nblk)
        def _():
            pltpu.sync_copy(stage, s_sh.at[pl.ds(j * sb, sb)])
        return 0
    lax.fori_loop(0, nown, zcp, 0)
    plsc.subcore_barrier()

    pltpu.sync_copy(idxc_hbm.at[wid], idxc)

    def chunk(j, _):
        def pair(q, _):
            pltpu.async_copy(
                ab_hbm.at[q].at[idxc.at[2 * j + q]],
                mbuf.at[pl.ds(q * CHUNK, CHUNK)],
                sem,
            ).wait()
            return 0
        lax.fori_loop(0, 2, pair, 0)

        def ew(e, _):
            for c in range(D // 16):
                sl = pl.ds(c * 16, 16)
                x = mbuf[e, sl] + mbuf[CHUNK + e, sl]
                neg = _SELU_ALPHA * (jnp.exp(jnp.minimum(x, 0.0)) - 1.0)
                mbuf[e, sl] = jnp.where(x > 0.0, x, neg) * _SELU_SCALE
            return 0
        lax.fori_loop(0, CHUNK, ew, 0)
        pltpu.sync_copy(
            mbuf.at[pl.ds(0, CHUNK)], s_sh.at[idxc.at[2 * j + 1]], add=True
        )
        return 0
    lax.fori_loop(0, nchunk, chunk, 0)
    plsc.subcore_barrier()

    def ocp(i, _):
        j = i * NS + sid

        @pl.when(j < nblk)
        def _():
            rows = pl.ds(j * sb, sb)
            pltpu.sync_copy(s_sh.at[rows], stage)
            pltpu.sync_copy(stage, out_hbm.at[cid, rows])
        return 0
    lax.fori_loop(0, nown, ocp, 0)


def _edge_call(ab, idxc):
    n = ab.shape[1]
    nchunk2 = idxc.shape[1]
    mesh = plsc.VectorSubcoreMesh(
        core_axis_name="c", subcore_axis_name="s", num_cores=NC, num_subcores=NS
    )
    fn = pl.kernel(
        _edge_body,
        out_type=jax.ShapeDtypeStruct((NC, n, D), jnp.float32),
        mesh=mesh,
        compiler_params=pltpu.CompilerParams(use_tc_tiling_on_sc=False),
        scratch_types=[
            pltpu.VMEM((nchunk2, CHUNK), jnp.int32),
            pltpu.VMEM((2 * CHUNK, D), jnp.float32),
            pltpu.VMEM((80, D), jnp.float32),
            pltpu.VMEM_SHARED((n, D), jnp.float32),
            pltpu.SemaphoreType.DMA,
        ],
    )
    return fn(ab, idxc)


def kernel(link_state, W_msg, b_msg, W_gru, U_gru, b_gru_in, b_gru_rec,
           W_r1, b_r1, W_r2, b_r2, W_r3, b_r3,
           states_graph_ids, states_first, states_second, sates_num_edges):
    n = link_state.shape[0]
    e = states_first.shape[0]
    assert e % (NW * CHUNK) == 0 and n % NS == 0

    w1 = W_msg[:D]
    w2 = W_msg[D:]
    bm = b_msg.reshape(1, D)
    bi = b_gru_in.reshape(1, 3 * D)
    br = b_gru_rec.reshape(1, 3 * D)
    perm = jnp.argsort(states_second, stable=True)
    sf = jnp.take(states_first, perm)
    ss = jnp.take(states_second, perm)

    nchunk = e // (NW * CHUNK)
    f3d = sf.reshape(NW, nchunk, 1, CHUNK)
    s3d = ss.reshape(NW, nchunk, 1, CHUNK)
    idxc = jnp.concatenate([f3d, s3d], axis=2).reshape(NW, 2 * nchunk, CHUNK)
    gid = states_graph_ids.reshape(1, n)

    h = link_state
    for _ in range(T):
        ab = _ab_call(h, w1, w2, bm)
        s = _edge_call(ab, idxc)
        h = _gru_call(s, h, W_gru, U_gru, bi, br)

    return _readout_call(
        h, gid,
        W_r1, b_r1.reshape(1, RU),
        W_r2, b_r2.reshape(1, RU),
        W_r3.reshape(1, RU), b_r3.reshape(1, 1),
    )


# speedup vs baseline: 1.9925x; 1.1903x over previous
import functools

import jax
import jax.numpy as jnp
from jax import lax
from jax.experimental import pallas as pl
from jax.experimental.pallas import tpu as pltpu
from jax.experimental.pallas import tpu_sc as plsc

D = 128
T = 4
N_GRAPHS = 64
RU = 256

NC, NS = 2, 16
NW = NC * NS
CHUNK = 80

_SELU_ALPHA = 1.6732632423543772
_SELU_SCALE = 1.0507009873554805


def _selu(x):
    neg = _SELU_SCALE * _SELU_ALPHA * (jnp.exp(jnp.minimum(x, 0.0)) - 1.0)
    return jnp.where(x > 0.0, _SELU_SCALE * x, neg)


def _ab_body(h_ref, w1_ref, w2_ref, bm_ref, ab_ref):
    h = h_ref[...]
    ab_ref[0] = jnp.dot(h, w1_ref[...], preferred_element_type=jnp.float32)
    ab_ref[1] = (
        jnp.dot(h, w2_ref[...], preferred_element_type=jnp.float32) + bm_ref[...]
    )


def _ab_call(h, w1, w2, bm):
    n = h.shape[0]
    blk = 1000
    grid = (n // blk,)
    return pl.pallas_call(
        _ab_body,
        grid=grid,
        in_specs=[
            pl.BlockSpec((blk, D), lambda i: (i, 0)),
            pl.BlockSpec((D, D), lambda i: (0, 0)),
            pl.BlockSpec((D, D), lambda i: (0, 0)),
            pl.BlockSpec((1, D), lambda i: (0, 0)),
        ],
        out_specs=pl.BlockSpec((2, blk, D), lambda i: (0, i, 0)),
        out_shape=jax.ShapeDtypeStruct((2, n, D), jnp.float32),
    )(h, w1, w2, bm)


def _gru_body(s_ref, h_ref, wg_ref, ug_ref, bi_ref, br_ref, o_ref):
    x = s_ref[0] + s_ref[1]
    h = h_ref[...]
    mx = jnp.dot(x, wg_ref[...], preferred_element_type=jnp.float32) + bi_ref[...]
    mh = jnp.dot(h, ug_ref[...], preferred_element_type=jnp.float32) + br_ref[...]
    z = jax.nn.sigmoid(mx[:, :D] + mh[:, :D])
    r = jax.nn.sigmoid(mx[:, D : 2 * D] + mh[:, D : 2 * D])
    cand = jnp.tanh(mx[:, 2 * D :] + r * mh[:, 2 * D :])
    o_ref[...] = z * h + (1.0 - z) * cand


def _gru_call(s, h, wg, ug, bi, br):
    n = h.shape[0]
    blk = 1000
    grid = (n // blk,)
    return pl.pallas_call(
        _gru_body,
        grid=grid,
        in_specs=[
            pl.BlockSpec((NC, blk, D), lambda i: (0, i, 0)),
            pl.BlockSpec((blk, D), lambda i: (i, 0)),
            pl.BlockSpec((D, 3 * D), lambda i: (0, 0)),
            pl.BlockSpec((D, 3 * D), lambda i: (0, 0)),
            pl.BlockSpec((1, 3 * D), lambda i: (0, 0)),
            pl.BlockSpec((1, 3 * D), lambda i: (0, 0)),
        ],
        out_specs=pl.BlockSpec((blk, D), lambda i: (i, 0)),
        out_shape=jax.ShapeDtypeStruct((n, D), jnp.float32),
    )(s, h, wg, ug, bi, br)


def _readout_body(h_ref, gid_ref, w1_ref, b1_ref, w2_ref, b2_ref, w3_ref, b3_ref, o_ref):
    n = h_ref.shape[0]
    ids = gid_ref[...]
    iota = lax.broadcasted_iota(jnp.int32, (N_GRAPHS, n), 0)
    mask = (ids == iota).astype(jnp.float32)
    gemb = jnp.dot(mask, h_ref[...], preferred_element_type=jnp.float32)
    r1 = _selu(jnp.dot(gemb, w1_ref[...], preferred_element_type=jnp.float32) + b1_ref[...])
    r2 = _selu(jnp.dot(r1, w2_ref[...], preferred_element_type=jnp.float32) + b2_ref[...])
    o_ref[...] = jnp.sum(r2 * w3_ref[...], axis=1, keepdims=True) + b3_ref[...]


def _readout_call(h, gid, w1, b1, w2, b2, w3t, b3):
    n = h.shape[0]
    return pl.pallas_call(
        _readout_body,
        in_specs=[
            pl.BlockSpec((n, D), lambda: (0, 0)),
            pl.BlockSpec((1, n), lambda: (0, 0)),
            pl.BlockSpec((D, RU), lambda: (0, 0)),
            pl.BlockSpec((1, RU), lambda: (0, 0)),
            pl.BlockSpec((RU, RU), lambda: (0, 0)),
            pl.BlockSpec((1, RU), lambda: (0, 0)),
            pl.BlockSpec((1, RU), lambda: (0, 0)),
            pl.BlockSpec((1, 1), lambda: (0, 0)),
        ],
        out_specs=pl.BlockSpec((N_GRAPHS, 1), lambda: (0, 0)),
        out_shape=jax.ShapeDtypeStruct((N_GRAPHS, 1), jnp.float32),
    )(h, gid, w1, b1, w2, b2, w3t, b3)


def _edge_body(ab_hbm, idxc_hbm, out_hbm, idxc, mbuf, stage, s_sh, sem):
    n = ab_hbm.shape[1]
    sb = stage.shape[0]
    nblk = n // sb
    nchunk = idxc.shape[0] // 2
    nown = (nblk + NS - 1) // NS

    cid = lax.axis_index("c")
    sid = lax.axis_index("s")
    wid = cid * NS + sid

    def zrow(i, _):
        for c in range(D // 16):
            stage[i, pl.ds(c * 16, 16)] = jnp.zeros((16,), jnp.float32)
        return 0
    lax.fori_loop(0, sb, zrow, 0)

    def zcp(i, _):
        j = i * NS + sid

        @pl.when(j < nblk)
        def _():
            pltpu.sync_copy(stage, s_sh.at[pl.ds(j * sb, sb)])
        return 0
    lax.fori_loop(0, nown, zcp, 0)
    plsc.subcore_barrier()

    pltpu.sync_copy(idxc_hbm.at[wid], idxc)

    def chunk(j, _):
        def pair(q, _):
            pltpu.async_copy(
                ab_hbm.at[q].at[idxc.at[2 * j + q]],
                mbuf.at[pl.ds(q * CHUNK, CHUNK)],
                sem,
            )
            return 0
        lax.fori_loop(0, 2, pair, 0)
        pltpu.make_async_copy(
            ab_hbm.at[0].at[pl.ds(0, 2 * CHUNK)], mbuf, sem
        ).wait()

        def ew(e, _):
            for c in range(D // 16):
                sl = pl.ds(c * 16, 16)
                x = mbuf[e, sl] + mbuf[CHUNK + e, sl]
                neg = _SELU_ALPHA * (jnp.exp(jnp.minimum(x, 0.0)) - 1.0)
                mbuf[e, sl] = jnp.where(x > 0.0, x, neg) * _SELU_SCALE
            return 0
        lax.fori_loop(0, CHUNK, ew, 0)
        pltpu.sync_copy(
            mbuf.at[pl.ds(0, CHUNK)], s_sh.at[idxc.at[2 * j + 1]], add=True
        )
        return 0
    lax.fori_loop(0, nchunk, chunk, 0)
    plsc.subcore_barrier()

    def ocp(i, _):
        j = i * NS + sid

        @pl.when(j < nblk)
        def _():
            rows = pl.ds(j * sb, sb)
            pltpu.sync_copy(s_sh.at[rows], stage)
            pltpu.sync_copy(stage, out_hbm.at[cid, rows])
        return 0
    lax.fori_loop(0, nown, ocp, 0)


def _edge_call(ab, idxc):
    n = ab.shape[1]
    nchunk2 = idxc.shape[1]
    mesh = plsc.VectorSubcoreMesh(
        core_axis_name="c", subcore_axis_name="s", num_cores=NC, num_subcores=NS
    )
    fn = pl.kernel(
        _edge_body,
        out_type=jax.ShapeDtypeStruct((NC, n, D), jnp.float32),
        mesh=mesh,
        compiler_params=pltpu.CompilerParams(use_tc_tiling_on_sc=False),
        scratch_types=[
            pltpu.VMEM((nchunk2, CHUNK), jnp.int32),
            pltpu.VMEM((2 * CHUNK, D), jnp.float32),
            pltpu.VMEM((80, D), jnp.float32),
            pltpu.VMEM_SHARED((n, D), jnp.float32),
            pltpu.SemaphoreType.DMA,
        ],
    )
    return fn(ab, idxc)


def kernel(link_state, W_msg, b_msg, W_gru, U_gru, b_gru_in, b_gru_rec,
           W_r1, b_r1, W_r2, b_r2, W_r3, b_r3,
           states_graph_ids, states_first, states_second, sates_num_edges):
    n = link_state.shape[0]
    e = states_first.shape[0]
    assert e % (NW * CHUNK) == 0 and n % NS == 0

    w1 = W_msg[:D]
    w2 = W_msg[D:]
    bm = b_msg.reshape(1, D)
    bi = b_gru_in.reshape(1, 3 * D)
    br = b_gru_rec.reshape(1, 3 * D)
    perm = jnp.argsort(states_second, stable=True)
    sf = jnp.take(states_first, perm)
    ss = jnp.take(states_second, perm)

    nchunk = e // (NW * CHUNK)
    f3d = sf.reshape(NW, nchunk, 1, CHUNK)
    s3d = ss.reshape(NW, nchunk, 1, CHUNK)
    idxc = jnp.concatenate([f3d, s3d], axis=2).reshape(NW, 2 * nchunk, CHUNK)
    gid = states_graph_ids.reshape(1, n)

    h = link_state
    for _ in range(T):
        ab = _ab_call(h, w1, w2, bm)
        s = _edge_call(ab, idxc)
        h = _gru_call(s, h, W_gru, U_gru, bi, br)

    return _readout_call(
        h, gid,
        W_r1, b_r1.reshape(1, RU),
        W_r2, b_r2.reshape(1, RU),
        W_r3.reshape(1, RU), b_r3.reshape(1, 1),
    )


# speedup vs baseline: 2.0590x; 1.0334x over previous
import functools

import jax
import jax.numpy as jnp
from jax import lax
from jax.experimental import pallas as pl
from jax.experimental.pallas import tpu as pltpu
from jax.experimental.pallas import tpu_sc as plsc

D = 128
T = 4
N_GRAPHS = 64
RU = 256

NC, NS = 2, 16
NW = NC * NS
CHUNK = 80

_SELU_ALPHA = 1.6732632423543772
_SELU_SCALE = 1.0507009873554805


def _selu(x):
    neg = _SELU_SCALE * _SELU_ALPHA * (jnp.exp(jnp.minimum(x, 0.0)) - 1.0)
    return jnp.where(x > 0.0, _SELU_SCALE * x, neg)


def _ab_body(h_ref, w1_ref, w2_ref, bm_ref, ab_ref):
    h = h_ref[...]
    ab_ref[0] = jnp.dot(h, w1_ref[...], preferred_element_type=jnp.float32)
    ab_ref[1] = (
        jnp.dot(h, w2_ref[...], preferred_element_type=jnp.float32) + bm_ref[...]
    )


def _ab_call(h, w1, w2, bm):
    n = h.shape[0]
    blk = 1000
    grid = (n // blk,)
    return pl.pallas_call(
        _ab_body,
        grid=grid,
        in_specs=[
            pl.BlockSpec((blk, D), lambda i: (i, 0)),
            pl.BlockSpec((D, D), lambda i: (0, 0)),
            pl.BlockSpec((D, D), lambda i: (0, 0)),
            pl.BlockSpec((1, D), lambda i: (0, 0)),
        ],
        out_specs=pl.BlockSpec((2, blk, D), lambda i: (0, i, 0)),
        out_shape=jax.ShapeDtypeStruct((2, n, D), jnp.float32),
    )(h, w1, w2, bm)


def _gru_body(s_ref, h_ref, wg_ref, ug_ref, bi_ref, br_ref, o_ref):
    x = s_ref[0] + s_ref[1]
    h = h_ref[...]
    mx = jnp.dot(x, wg_ref[...], preferred_element_type=jnp.float32) + bi_ref[...]
    mh = jnp.dot(h, ug_ref[...], preferred_element_type=jnp.float32) + br_ref[...]
    z = jax.nn.sigmoid(mx[:, :D] + mh[:, :D])
    r = jax.nn.sigmoid(mx[:, D : 2 * D] + mh[:, D : 2 * D])
    cand = jnp.tanh(mx[:, 2 * D :] + r * mh[:, 2 * D :])
    o_ref[...] = z * h + (1.0 - z) * cand


def _gru_call(s, h, wg, ug, bi, br):
    n = h.shape[0]
    blk = 1000
    grid = (n // blk,)
    return pl.pallas_call(
        _gru_body,
        grid=grid,
        in_specs=[
            pl.BlockSpec((NC, blk, D), lambda i: (0, i, 0)),
            pl.BlockSpec((blk, D), lambda i: (i, 0)),
            pl.BlockSpec((D, 3 * D), lambda i: (0, 0)),
            pl.BlockSpec((D, 3 * D), lambda i: (0, 0)),
            pl.BlockSpec((1, 3 * D), lambda i: (0, 0)),
            pl.BlockSpec((1, 3 * D), lambda i: (0, 0)),
        ],
        out_specs=pl.BlockSpec((blk, D), lambda i: (i, 0)),
        out_shape=jax.ShapeDtypeStruct((n, D), jnp.float32),
    )(s, h, wg, ug, bi, br)


def _readout_body(h_ref, gid_ref, w1_ref, b1_ref, w2_ref, b2_ref, w3_ref, b3_ref, o_ref):
    n = h_ref.shape[0]
    ids = gid_ref[...]
    iota = lax.broadcasted_iota(jnp.int32, (N_GRAPHS, n), 0)
    mask = (ids == iota).astype(jnp.float32)
    gemb = jnp.dot(mask, h_ref[...], preferred_element_type=jnp.float32)
    r1 = _selu(jnp.dot(gemb, w1_ref[...], preferred_element_type=jnp.float32) + b1_ref[...])
    r2 = _selu(jnp.dot(r1, w2_ref[...], preferred_element_type=jnp.float32) + b2_ref[...])
    o_ref[...] = jnp.sum(r2 * w3_ref[...], axis=1, keepdims=True) + b3_ref[...]


def _readout_call(h, gid, w1, b1, w2, b2, w3t, b3):
    n = h.shape[0]
    return pl.pallas_call(
        _readout_body,
        in_specs=[
            pl.BlockSpec((n, D), lambda: (0, 0)),
            pl.BlockSpec((1, n), lambda: (0, 0)),
            pl.BlockSpec((D, RU), lambda: (0, 0)),
            pl.BlockSpec((1, RU), lambda: (0, 0)),
            pl.BlockSpec((RU, RU), lambda: (0, 0)),
            pl.BlockSpec((1, RU), lambda: (0, 0)),
            pl.BlockSpec((1, RU), lambda: (0, 0)),
            pl.BlockSpec((1, 1), lambda: (0, 0)),
        ],
        out_specs=pl.BlockSpec((N_GRAPHS, 1), lambda: (0, 0)),
        out_shape=jax.ShapeDtypeStruct((N_GRAPHS, 1), jnp.float32),
    )(h, gid, w1, b1, w2, b2, w3t, b3)


def _edge_body(ab_hbm, idxc_hbm, out_hbm, idxc, mbuf, stage, s_sh, sem):
    n = ab_hbm.shape[1]
    sb = stage.shape[0]
    nblk = n // sb
    nchunk = idxc.shape[0] // 2
    nown = (nblk + NS - 1) // NS

    cid = lax.axis_index("c")
    sid = lax.axis_index("s")
    wid = cid * NS + sid

    def zrow(i, _):
        for c in range(D // 16):
            stage[i, pl.ds(c * 16, 16)] = jnp.zeros((16,), jnp.float32)
        return 0
    lax.fori_loop(0, sb, zrow, 0)

    def zcp(i, _):
        j = i * NS + sid

        @pl.when(j < nblk)
        def _():
            pltpu.sync_copy(stage, s_sh.at[pl.ds(j * sb, sb)])
        return 0
    lax.fori_loop(0, nown, zcp, 0)
    plsc.subcore_barrier()

    pltpu.sync_copy(idxc_hbm.at[wid], idxc)

    def chunk(j, _):
        def pair(q, _):
            pltpu.async_copy(
                ab_hbm.at[q].at[idxc.at[2 * j + q]],
                mbuf.at[pl.ds(q * CHUNK, CHUNK)],
                sem,
            )
            return 0
        lax.fori_loop(0, 2, pair, 0)
        pltpu.make_async_copy(
            ab_hbm.at[0].at[pl.ds(0, 2 * CHUNK)], mbuf, sem
        ).wait()

        def ew(i, _):
            for u in range(2):
                e = i * 2 + u
                for c in range(D // 16):
                    sl = pl.ds(c * 16, 16)
                    x = mbuf[e, sl] + mbuf[CHUNK + e, sl]
                    neg = _SELU_ALPHA * (jnp.exp(jnp.minimum(x, 0.0)) - 1.0)
                    mbuf[e, sl] = jnp.where(x > 0.0, x, neg) * _SELU_SCALE
            return 0
        lax.fori_loop(0, CHUNK // 2, ew, 0)
        pltpu.sync_copy(
            mbuf.at[pl.ds(0, CHUNK)], s_sh.at[idxc.at[2 * j + 1]], add=True
        )
        return 0
    lax.fori_loop(0, nchunk, chunk, 0)
    plsc.subcore_barrier()

    def ocp(i, _):
        j = i * NS + sid

        @pl.when(j < nblk)
        def _():
            rows = pl.ds(j * sb, sb)
            pltpu.sync_copy(s_sh.at[rows], stage)
            pltpu.sync_copy(stage, out_hbm.at[cid, rows])
        return 0
    lax.fori_loop(0, nown, ocp, 0)


def _edge_call(ab, idxc):
    n = ab.shape[1]
    nchunk2 = idxc.shape[1]
    mesh = plsc.VectorSubcoreMesh(
        core_axis_name="c", subcore_axis_name="s", num_cores=NC, num_subcores=NS
    )
    fn = pl.kernel(
        _edge_body,
        out_type=jax.ShapeDtypeStruct((NC, n, D), jnp.float32),
        mesh=mesh,
        compiler_params=pltpu.CompilerParams(use_tc_tiling_on_sc=False),
        scratch_types=[
            pltpu.VMEM((nchunk2, CHUNK), jnp.int32),
            pltpu.VMEM((2 * CHUNK, D), jnp.float32),
            pltpu.VMEM((80, D), jnp.float32),
            pltpu.VMEM_SHARED((n, D), jnp.float32),
            pltpu.SemaphoreType.DMA,
        ],
    )
    return fn(ab, idxc)


def kernel(link_state, W_msg, b_msg, W_gru, U_gru, b_gru_in, b_gru_rec,
           W_r1, b_r1, W_r2, b_r2, W_r3, b_r3,
           states_graph_ids, states_first, states_second, sates_num_edges):
    n = link_state.shape[0]
    e = states_first.shape[0]
    assert e % (NW * CHUNK) == 0 and n % NS == 0

    w1 = W_msg[:D]
    w2 = W_msg[D:]
    bm = b_msg.reshape(1, D)
    bi = b_gru_in.reshape(1, 3 * D)
    br = b_gru_rec.reshape(1, 3 * D)
    perm = jnp.argsort(states_second, stable=True)
    sf = jnp.take(states_first, perm)
    ss = jnp.take(states_second, perm)

    nchunk = e // (NW * CHUNK)
    f3d = sf.reshape(NW, nchunk, 1, CHUNK)
    s3d = ss.reshape(NW, nchunk, 1, CHUNK)
    idxc = jnp.concatenate([f3d, s3d], axis=2).reshape(NW, 2 * nchunk, CHUNK)
    gid = states_graph_ids.reshape(1, n)

    h = link_state
    for _ in range(T):
        ab = _ab_call(h, w1, w2, bm)
        s = _edge_call(ab, idxc)
        h = _gru_call(s, h, W_gru, U_gru, bi, br)

    return _readout_call(
        h, gid,
        W_r1, b_r1.reshape(1, RU),
        W_r2, b_r2.reshape(1, RU),
        W_r3.reshape(1, RU), b_r3.reshape(1, 1),
    )
